# baseline (speedup 1.0000x reference)
import functools

import jax
import jax.numpy as jnp
from jax import lax
from jax.experimental import pallas as pl
from jax.experimental.pallas import tpu as pltpu
from jax.experimental.pallas import tpu_sc as plsc

D = 128
S = 9
O = 128

NC = 2
NS = 16
L = 16
NW = NC * NS

CH = 40
K0 = 61
K1 = 19
CORE0_TOTAL = NS * K0 * CH
NPAD = NS * (K0 + K1) * CH
ROWS = CH * S
GR = 120
NG = ROWS // GR
IDX_MAX = max(K0, K1) * CH * S
FCHUNKS = -(-IDX_MAX // 16)

BR = 25088
NBLK = 2
NP = NBLK * BR


def _matmul_body(x_ref, w_ref, z_ref):
    z_ref[...] = jnp.dot(x_ref[...], w_ref[...],
                         preferred_element_type=jnp.float32)


def _tc_matmul(x, wcat):
    return pl.pallas_call(
        _matmul_body,
        grid=(NBLK, S),
        in_specs=[
            pl.BlockSpec((BR, D), lambda i, s: (i, 0)),
            pl.BlockSpec((D, O), lambda i, s: (0, s)),
        ],
        out_specs=pl.BlockSpec((BR, O), lambda i, s: (s * NBLK + i, 0)),
        out_shape=jax.ShapeDtypeStruct((S * NP, O), jnp.float32),
    )(x, wcat)


def _sc_body(z_ref, idx_ref, b_ref, out_ref,
             fbuf, gb0, gb1, ob0, ob1, bbuf, semA, semB, osemA, osemB):
    cid = lax.axis_index("c")
    sid = lax.axis_index("s")
    base = jnp.where(cid == 0, NS * (K1 * CH) + sid * (K0 * CH),
                     sid * (K1 * CH))
    nch = jnp.where(cid == 0, K0, K1)
    bufA = (gb0, ob0, semA, osemA)
    bufB = (gb1, ob1, semB, osemB)

    pltpu.sync_copy(b_ref, bbuf)
    bvecs = [bbuf[pl.ds(p * L, L)] for p in range(O // L)]

    pltpu.sync_copy(idx_ref.at[pl.ds(base * S, IDX_MAX)],
                    fbuf.at[pl.ds(0, IDX_MAX)])

    def flat_body(c, carry):
        jv = lax.iota(jnp.int32, L) + c * L
        sv = lax.rem(jv, S)
        fbuf[pl.ds(c * L, L)] = sv * NP + fbuf[pl.ds(c * L, L)]
        return carry

    lax.fori_loop(0, FCHUNKS, flat_body, 0)

    def fire(k, buf):
        gbuf = buf[0]
        for g in range(NG):
            pltpu.async_copy(
                z_ref.at[fbuf.at[pl.ds(k * ROWS + g * GR, GR)]],
                gbuf.at[pl.ds(g * GR, GR)], buf[2])

    def drain_acc_store(k, buf):
        gbuf, obuf, sem, osem = buf

        @pl.when(jnp.logical_and(k >= 2, k < nch))
        def _():
            pltpu.make_async_copy(
                obuf, out_ref.at[pl.ds(base, CH)], osem).wait()

        for g in range(NG):
            pltpu.make_async_copy(
                z_ref.at[fbuf.at[pl.ds(k * ROWS + g * GR, GR)]],
                gbuf.at[pl.ds(g * GR, GR)], sem).wait()

        @plsc.parallel_loop(0, CH, 1, unroll=2)
        def acc_body(n):
            accs = list(bvecs)
            for s in range(S):
                row = n * S + s
                for p in range(O // L):
                    accs[p] = accs[p] + gbuf[row, pl.ds(p * L, L)]
            for p in range(O // L):
                obuf[n, pl.ds(p * L, L)] = accs[p]

        pltpu.async_copy(obuf, out_ref.at[pl.ds(base + k * CH, CH)], osem)

    fire(0, bufA)

    def guarded(cond, fn, *args):
        @pl.when(cond)
        def _():
            fn(*args)

    def pair_body(t, carry):
        k0 = 2 * t
        guarded(k0 + 1 < nch, fire, k0 + 1, bufB)
        guarded(k0 < nch, drain_acc_store, k0, bufA)
        guarded(k0 + 2 < nch, fire, k0 + 2, bufA)
        guarded(k0 + 1 < nch, drain_acc_store, k0 + 1, bufB)
        return carry

    lax.fori_loop(0, (max(K0, K1) + 1) // 2, pair_body, 0)

    pltpu.make_async_copy(ob0, out_ref.at[pl.ds(base, CH)], osemA).wait()
    pltpu.make_async_copy(ob1, out_ref.at[pl.ds(base, CH)], osemB).wait()


_sc_gather_sum = functools.partial(
    pl.kernel,
    out_type=jax.ShapeDtypeStruct((NPAD, O), jnp.float32),
    mesh=plsc.VectorSubcoreMesh(core_axis_name="c", subcore_axis_name="s",
                                num_cores=NC, num_subcores=NS),
    scratch_types=(
        [pltpu.VMEM((FCHUNKS * 16,), jnp.int32)]
        + [pltpu.VMEM((ROWS, O), jnp.float32)] * 2
        + [pltpu.VMEM((CH, O), jnp.float32)] * 2
        + [pltpu.VMEM((O,), jnp.float32),
           pltpu.SemaphoreType.DMA,
           pltpu.SemaphoreType.DMA,
           pltpu.SemaphoreType.DMA,
           pltpu.SemaphoreType.DMA]
    ),
)(_sc_body)


def kernel(x, indices, W, b):
    n_nodes = x.shape[0]
    wcat = W.reshape(S, D, O).transpose(1, 0, 2).reshape(D, S * O)
    z = _tc_matmul(x.astype(jnp.bfloat16), wcat.astype(jnp.bfloat16))
    idx_pad = jnp.pad(indices, ((0, NPAD - n_nodes + 8), (0, 0))).reshape(-1)
    out = _sc_gather_sum(z, idx_pad.astype(jnp.int32), b)
    return out[:n_nodes]


# speedup vs baseline: 1.1615x; 1.1615x over previous
import functools

import jax
import jax.numpy as jnp
from jax import lax
from jax.experimental import pallas as pl
from jax.experimental.pallas import tpu as pltpu
from jax.experimental.pallas import tpu_sc as plsc

D = 128
S = 9
O = 128

NC = 2
NS = 16
L = 16
NW = NC * NS

CH = 40
K0 = 61
K1 = 19
CORE0_TOTAL = NS * K0 * CH
NPAD = NS * (K0 + K1) * CH
ROWS = CH * S
GR = 120
NG = ROWS // GR
IDX_MAX = max(K0, K1) * CH * S
FCHUNKS = -(-IDX_MAX // 16)

BR = 25088
NBLK = 2
NP = NBLK * BR


def _matmul_body(x_ref, w_ref, z_ref):
    z_ref[...] = jnp.dot(x_ref[...], w_ref[...],
                         preferred_element_type=jnp.float32)


def _tc_matmul(x, wcat):
    return pl.pallas_call(
        _matmul_body,
        grid=(NBLK, S),
        in_specs=[
            pl.BlockSpec((BR, D), lambda i, s: (i, 0)),
            pl.BlockSpec((D, O), lambda i, s: (0, s)),
        ],
        out_specs=pl.BlockSpec((BR, O), lambda i, s: (s * NBLK + i, 0)),
        out_shape=jax.ShapeDtypeStruct((S * NP, O), jnp.float32),
    )(x, wcat)


def _sc_body(z_ref, idx_ref, b_ref, out_ref,
             fbuf, gb0, gb1, ob0, ob1, bbuf, semA, semB, osemA, osemB):
    cid = lax.axis_index("c")
    sid = lax.axis_index("s")
    base = jnp.where(cid == 0, sid * (K0 * CH), CORE0_TOTAL + sid * (K1 * CH))
    nch = jnp.where(cid == 0, K0, K1)
    bufA = (gb0, ob0, semA, osemA)
    bufB = (gb1, ob1, semB, osemB)

    pltpu.sync_copy(b_ref, bbuf)
    bvecs = [bbuf[pl.ds(p * L, L)] for p in range(O // L)]

    @pl.when(cid == 0)
    def _():
        pltpu.sync_copy(idx_ref.at[pl.ds(base * S, K0 * CH * S)],
                        fbuf.at[pl.ds(0, K0 * CH * S)])

    @pl.when(cid == 1)
    def _():
        pltpu.sync_copy(idx_ref.at[pl.ds(base * S, K1 * CH * S)],
                        fbuf.at[pl.ds(0, K1 * CH * S)])

    def flat_body(c, carry):
        jv = lax.iota(jnp.int32, L) + c * L
        sv = lax.rem(jv, S)
        fbuf[pl.ds(c * L, L)] = sv * NP + fbuf[pl.ds(c * L, L)]
        return carry

    lax.fori_loop(0, FCHUNKS, flat_body, 0)

    def fire(k, buf):
        gbuf = buf[0]
        for g in range(NG):
            pltpu.async_copy(
                z_ref.at[fbuf.at[pl.ds(k * ROWS + g * GR, GR)]],
                gbuf.at[pl.ds(g * GR, GR)], buf[2])

    def drain_acc_store(k, buf):
        gbuf, obuf, sem, osem = buf

        @pl.when(jnp.logical_and(k >= 2, k < nch))
        def _():
            pltpu.make_async_copy(
                obuf, out_ref.at[pl.ds(base, CH)], osem).wait()

        for g in range(NG):
            pltpu.make_async_copy(
                z_ref.at[fbuf.at[pl.ds(k * ROWS + g * GR, GR)]],
                gbuf.at[pl.ds(g * GR, GR)], sem).wait()

        @plsc.parallel_loop(0, CH, 1, unroll=2)
        def acc_body(n):
            accs = list(bvecs)
            for s in range(S):
                row = n * S + s
                for p in range(O // L):
                    accs[p] = accs[p] + gbuf[row, pl.ds(p * L, L)]
            for p in range(O // L):
                obuf[n, pl.ds(p * L, L)] = accs[p]

        pltpu.async_copy(obuf, out_ref.at[pl.ds(base + k * CH, CH)], osem)

    fire(0, bufA)

    def guarded(cond, fn, *args):
        @pl.when(cond)
        def _():
            fn(*args)

    def pair_body(t, carry):
        k0 = 2 * t
        guarded(k0 + 1 < nch, fire, k0 + 1, bufB)
        guarded(k0 < nch, drain_acc_store, k0, bufA)
        guarded(k0 + 2 < nch, fire, k0 + 2, bufA)
        guarded(k0 + 1 < nch, drain_acc_store, k0 + 1, bufB)
        return carry

    lax.fori_loop(0, (max(K0, K1) + 1) // 2, pair_body, 0)

    pltpu.make_async_copy(ob0, out_ref.at[pl.ds(base, CH)], osemA).wait()
    pltpu.make_async_copy(ob1, out_ref.at[pl.ds(base, CH)], osemB).wait()


_sc_gather_sum = functools.partial(
    pl.kernel,
    out_type=jax.ShapeDtypeStruct((NPAD, O), jnp.float32),
    mesh=plsc.VectorSubcoreMesh(core_axis_name="c", subcore_axis_name="s",
                                num_cores=NC, num_subcores=NS),
    scratch_types=(
        [pltpu.VMEM((FCHUNKS * 16,), jnp.int32)]
        + [pltpu.VMEM((ROWS, O), jnp.float32)] * 2
        + [pltpu.VMEM((CH, O), jnp.float32)] * 2
        + [pltpu.VMEM((O,), jnp.float32),
           pltpu.SemaphoreType.DMA,
           pltpu.SemaphoreType.DMA,
           pltpu.SemaphoreType.DMA,
           pltpu.SemaphoreType.DMA]
    ),
)(_sc_body)


def kernel(x, indices, W, b):
    n_nodes = x.shape[0]
    wcat = W.reshape(S, D, O).transpose(1, 0, 2).reshape(D, S * O)
    z = _tc_matmul(x.astype(jnp.bfloat16), wcat.astype(jnp.bfloat16))
    idx_pad = jnp.pad(indices, ((0, NPAD - n_nodes + 8), (0, 0))).reshape(-1)
    out = _sc_gather_sum(z, idx_pad.astype(jnp.int32), b)
    return out[:n_nodes]
